# baseline (speedup 1.0000x reference)
import functools

import jax
import jax.numpy as jnp
from jax import lax
from jax.experimental import pallas as pl
from jax.experimental.pallas import tpu as pltpu
from jax.experimental.pallas import tpu_sc as plsc

NUM_NODES = 1000000
MEMORY_DIM = 64
BATCH = 16384

_NC = 2
_NS = 16
_NW = _NC * _NS
_BPW = BATCH // _NW
_CHUNK = 128
_NCHUNK = _BPW // _CHUNK
_L = 16

_mesh = plsc.VectorSubcoreMesh(core_axis_name="c", subcore_axis_name="s")


@functools.partial(
    pl.kernel,
    mesh=_mesh,
    out_type=[
        jax.ShapeDtypeStruct((BATCH * MEMORY_DIM,), jnp.float32),
        jax.ShapeDtypeStruct((BATCH,), jnp.int32),
    ],
    scratch_types=[
        pltpu.VMEM((_BPW,), jnp.int32),
        pltpu.VMEM((_NCHUNK, _CHUNK), jnp.int32),
        pltpu.VMEM((_BPW, 2 * MEMORY_DIM), jnp.float32),
        pltpu.VMEM((_BPW * MEMORY_DIM,), jnp.float32),
        pltpu.VMEM((_BPW,), jnp.int32),
        pltpu.SemaphoreType.DMA,
        pltpu.SemaphoreType.DMA,
    ],
)
def _dual_gather(mem2_hbm, lu_hbm, idx_hbm, mem_out, lu_out,
                 idx_v, pair_v, rows2_v, rows_v, lu_v,
                 sem_rows, sem_lu):
    wid = lax.axis_index("s") * _NC + lax.axis_index("c")
    base = wid * _BPW
    pltpu.sync_copy(idx_hbm.at[pl.ds(base, _BPW)], idx_v)
    for j in range(_NCHUNK):
        for v in range(_CHUNK // _L):
            pair_v[j, pl.ds(v * _L, _L)] = (
                idx_v[pl.ds(j * _CHUNK + v * _L, _L)] >> 1)
    copies = []
    for j in range(_NCHUNK):
        copies.append(pltpu.async_copy(
            mem2_hbm.at[pair_v.at[j]],
            rows2_v.at[pl.ds(j * _CHUNK, _CHUNK)],
            sem_rows))
        copies.append(pltpu.async_copy(
            lu_hbm.at[idx_v.at[pl.ds(j * _CHUNK, _CHUNK)]],
            lu_v.at[pl.ds(j * _CHUNK, _CHUNK)],
            sem_lu))
    for c in copies:
        c.wait()

    def _pick(g, _):
        vec = idx_v[pl.ds(g * _L, _L)]
        for l in range(_L):
            off = (vec[l] & 1) * MEMORY_DIM
            r = g * _L + l
            for c in range(MEMORY_DIM // _L):
                rows_v[pl.ds(r * MEMORY_DIM + c * _L, _L)] = (
                    rows2_v[r, pl.ds(off + c * _L, _L)])
        return _

    lax.fori_loop(0, _BPW // _L, _pick, 0)

    pltpu.sync_copy(rows_v, mem_out.at[pl.ds(base * MEMORY_DIM, _BPW * MEMORY_DIM)])
    pltpu.sync_copy(lu_v, lu_out.at[pl.ds(base, _BPW)])


def kernel(memory, last_update, n_id):
    mem2 = memory.reshape(NUM_NODES // 2, 2 * MEMORY_DIM)
    idx = n_id.astype(jnp.int32)
    lu32 = last_update.astype(jnp.int32)
    mem_flat, lu_out = _dual_gather(mem2, lu32, idx)
    return (mem_flat.reshape(BATCH, MEMORY_DIM), lu_out.astype(last_update.dtype))


# speedup vs baseline: 1.5808x; 1.5808x over previous
import functools

import jax
import jax.numpy as jnp
from jax import lax
from jax.experimental import pallas as pl
from jax.experimental.pallas import tpu as pltpu
from jax.experimental.pallas import tpu_sc as plsc

NUM_NODES = 1000000
MEMORY_DIM = 64
BATCH = 16384

_NC = 2
_NS = 16
_NW = _NC * _NS
_BPW = BATCH // _NW
_TCH = 64
_NTCH = _BPW // _TCH
_CHUNK = 128
_NCHUNK = _BPW // _CHUNK
_L = 16

_mesh = plsc.VectorSubcoreMesh(core_axis_name="c", subcore_axis_name="s")


@functools.partial(
    pl.kernel,
    mesh=_mesh,
    out_type=[
        jax.ShapeDtypeStruct((BATCH * MEMORY_DIM,), jnp.float32),
        jax.ShapeDtypeStruct((BATCH,), jnp.int32),
    ],
    scratch_types=[
        pltpu.VMEM((_BPW,), jnp.int32),
        pltpu.VMEM((_TCH, 8, MEMORY_DIM), jnp.float32),
        pltpu.VMEM((_BPW * MEMORY_DIM,), jnp.float32),
        pltpu.VMEM((_BPW,), jnp.int32),
        pltpu.SemaphoreType.DMA,
        pltpu.SemaphoreType.DMA,
    ],
)
def _dual_gather(mem_hbm, lu_hbm, idx_hbm, mem_out, lu_out,
                 idx_v, tiles_v, rows_v, lu_v, sem_tiles, sem_lu):
    wid = lax.axis_index("s") * _NC + lax.axis_index("c")
    base = wid * _BPW
    pltpu.sync_copy(idx_hbm.at[pl.ds(base, _BPW)], idx_v)

    lu_copies = []
    for j in range(_NCHUNK):
        lu_copies.append(pltpu.async_copy(
            lu_hbm.at[idx_v.at[pl.ds(j * _CHUNK, _CHUNK)]],
            lu_v.at[pl.ds(j * _CHUNK, _CHUNK)],
            sem_lu))

    def _chunk(t, _):
        cbase = t * _TCH
        for v in range(_TCH // _L):
            vec = idx_v[pl.ds(cbase + v * _L, _L)]
            for l in range(_L):
                tile = (vec[l] >> 3) * 8
                pltpu.async_copy(mem_hbm.at[pl.ds(tile, 8), :],
                                 tiles_v.at[v * _L + l],
                                 sem_tiles)
        pltpu.make_async_copy(
            mem_hbm.at[pl.ds(0, _TCH * 8), :].reshape(_TCH, 8, MEMORY_DIM),
            tiles_v, sem_tiles).wait()
        for v in range(_TCH // _L):
            vec = idx_v[pl.ds(cbase + v * _L, _L)]
            for l in range(_L):
                q = v * _L + l
                sub = vec[l] & 7
                dst0 = (cbase + q) * MEMORY_DIM
                for c in range(MEMORY_DIM // _L):
                    rows_v[pl.ds(dst0 + c * _L, _L)] = (
                        tiles_v[q, sub, pl.ds(c * _L, _L)])
        return _

    lax.fori_loop(0, _NTCH, _chunk, 0)

    for c in lu_copies:
        c.wait()

    pltpu.sync_copy(rows_v, mem_out.at[pl.ds(base * MEMORY_DIM, _BPW * MEMORY_DIM)])
    pltpu.sync_copy(lu_v, lu_out.at[pl.ds(base, _BPW)])


def kernel(memory, last_update, n_id):
    idx = n_id.astype(jnp.int32)
    lu32 = last_update.astype(jnp.int32)
    mem_flat, lu_out = _dual_gather(memory, lu32, idx)
    return (mem_flat.reshape(BATCH, MEMORY_DIM), lu_out.astype(last_update.dtype))
